# Initial kernel scaffold; baseline (speedup 1.0000x reference)
import jax
import jax.numpy as jnp
from jax.experimental import pallas as pl


def kernel(x, edge_index, edge_attr, batch, emb, W1, b1, W2, b2, W3, b3):
    raise NotImplementedError("write your pallas kernel here")


# speedup vs baseline: 7.4521x; 7.4521x over previous
import functools

import jax
import jax.numpy as jnp
from jax import lax
from jax.experimental import pallas as pl
from jax.experimental.pallas import tpu as pltpu
from jax.experimental.pallas import tpu_sc as plsc

N = 10000
E = 320000
D = 128
G = 16
T = 30
TP = 32
NP = 10240
NC = 2
NS = 16
NW = NC * NS
EPT = E // NW
CH = 128
KC = 80
EPTP = KC * CH
NPT = NP // NS
GRP = 8
SB = 16
NSB = KC // SB

_mesh = plsc.VectorSubcoreMesh(
    core_axis_name="c", subcore_axis_name="s", num_cores=NC, num_subcores=NS)
_sc_params = pltpu.CompilerParams(needs_layout_passes=False)

_f32 = jnp.float32
_i32 = jnp.int32


def _wid():
    cid = lax.axis_index("c")
    sid = lax.axis_index("s")
    return cid, sid, cid * NS + sid


@functools.partial(
    pl.kernel,
    out_type=jax.ShapeDtypeStruct((NC, NP), _f32),
    mesh=_mesh,
    compiler_params=_sc_params,
    scratch_types=[
        pltpu.VMEM((KC, CH), _i32),
        pltpu.VMEM((KC, CH), _f32),
        pltpu.VMEM((NPT,), _f32),
        pltpu.VMEM_SHARED((NP,), _f32),
        pltpu.SemaphoreType.DMA,
    ],
)
def _deg_kernel(dst3, ew3, degp, dstv, ewv, zbuf, acc, sem):
    cid, sid, wid = _wid()
    pltpu.sync_copy(dst3.at[wid], dstv)
    pltpu.sync_copy(ew3.at[wid], ewv)

    def _z(i, _):
        zbuf[pl.ds(i * 16, 16)] = jnp.zeros((16,), _f32)
        return 0
    lax.fori_loop(0, NPT // 16, _z, 0)
    pltpu.sync_copy(zbuf, acc.at[pl.ds(sid * NPT, NPT)])
    plsc.subcore_barrier()

    def _grp(g, _):
        descs = []
        for j in range(GRP):
            k = g * GRP + j
            descs.append(
                pltpu.async_copy(ewv.at[k], acc.at[dstv.at[k]], sem, add=True))
        for d in descs:
            d.wait()
        return 0
    lax.fori_loop(0, KC // GRP, _grp, 0)
    plsc.subcore_barrier()
    pltpu.sync_copy(acc.at[pl.ds(sid * NPT, NPT)],
                    degp.at[cid, pl.ds(sid * NPT, NPT)])


@functools.partial(
    pl.kernel,
    out_type=jax.ShapeDtypeStruct((NW, KC, CH), _f32),
    mesh=_mesh,
    compiler_params=_sc_params,
    scratch_types=[
        pltpu.VMEM((NP,), _f32),
        pltpu.VMEM((KC, CH), _i32),
        pltpu.VMEM((KC, CH), _i32),
        pltpu.VMEM((KC, CH), _f32),
        pltpu.VMEM((KC, CH), _f32),
    ],
)
def _norm_kernel(src3, dst3, ew3, dinv, norm3, dv, srcv, dstv, ewv, nv):
    _, _, wid = _wid()
    pltpu.sync_copy(dinv, dv)
    pltpu.sync_copy(src3.at[wid], srcv)
    pltpu.sync_copy(dst3.at[wid], dstv)
    pltpu.sync_copy(ew3.at[wid], ewv)

    def _chunk(k, _):
        for j in range(CH // 16):
            sl = pl.ds(j * 16, 16)
            si = srcv[k, sl]
            di = dstv[k, sl]
            w = ewv[k, sl]
            a = plsc.load_gather(dv, [si])
            b = plsc.load_gather(dv, [di])
            nv[k, sl] = a * w * b
        return 0
    lax.fori_loop(0, KC, _chunk, 0)
    pltpu.sync_copy(nv, norm3.at[wid])


@functools.partial(
    pl.kernel,
    out_type=jax.ShapeDtypeStruct((NC, NP, D), _f32),
    mesh=_mesh,
    compiler_params=_sc_params,
    scratch_types=[
        pltpu.VMEM((SB, CH), _i32),
        pltpu.VMEM((SB, CH), _i32),
        pltpu.VMEM((SB, CH), _f32),
        pltpu.VMEM((2, CH, D), _f32),
        pltpu.VMEM_SHARED((NP, D), _f32),
        pltpu.SemaphoreType.DMA,
        pltpu.SemaphoreType.DMA,
        pltpu.SemaphoreType.DMA,
        pltpu.SemaphoreType.DMA,
    ],
)
def _mp_kernel(table, src3, dst3, norm3, msgp,
               srcb, dstb, nvb, rows, acc, semg0, semg1, sems0, sems1):
    cid, sid, wid = _wid()
    semg = (semg0, semg1)
    sems = (sems0, sems1)

    def _z(i, _):
        for j in range(D // 16):
            rows[0, i, pl.ds(j * 16, 16)] = jnp.zeros((16,), _f32)
        return 0
    lax.fori_loop(0, CH, _z, 0)

    def _zacc(i, _):
        pltpu.sync_copy(rows.at[0], acc.at[pl.ds(sid * NPT + i * CH, CH)])
        return 0
    lax.fori_loop(0, NPT // CH, _zacc, 0)
    plsc.subcore_barrier()

    def _gather(c, p):
        pltpu.async_copy(table.at[srcb.at[c]], rows.at[p], semg[p])

    def _gather_wait(c, p):
        pltpu.make_async_copy(table.at[srcb.at[c]], rows.at[p],
                              semg[p]).wait()

    def _scatter(c, p):
        pltpu.async_copy(rows.at[p], acc.at[dstb.at[c]], sems[p], add=True)

    def _scatter_wait(c, p):
        pltpu.make_async_copy(rows.at[p], acc.at[dstb.at[c]],
                              sems[p]).wait()

    def _scale(c, p):
        def _row(r, _):
            s = plsc.load_gather(
                nvb, [jnp.broadcast_to(c, (16,)), jnp.broadcast_to(r, (16,))])
            for j in range(D // 16):
                sl = pl.ds(j * 16, 16)
                rows[p, r, sl] = rows[p, r, sl] * s
            return 0
        lax.fori_loop(0, CH, _row, 0)

    for b in range(NSB):
        pltpu.sync_copy(src3.at[wid, pl.ds(b * SB, SB)], srcb)
        pltpu.sync_copy(dst3.at[wid, pl.ds(b * SB, SB)], dstb)
        pltpu.sync_copy(norm3.at[wid, pl.ds(b * SB, SB)], nvb)
        _gather(0, 0)

        def _two(i, _):
            for p in range(2):
                c = i * 2 + p

                @pl.when(c > 0)
                def _():
                    _scatter_wait(c - 1, 1 - p)

                @pl.when(c < SB - 1)
                def _():
                    _gather(c + 1, 1 - p)
                _gather_wait(c, p)
                _scale(c, p)
                _scatter(c, p)
            return 0
        lax.fori_loop(0, SB // 2, _two, 0)
        _scatter_wait(SB - 1, 1)

    plsc.subcore_barrier()
    pltpu.sync_copy(acc.at[pl.ds(sid * NPT, NPT)],
                    msgp.at[cid, pl.ds(sid * NPT, NPT)])


def _prep_body(degp_ref, x_ref, embp_ref, w1_ref, dinv_ref, sn_ref, h1w_ref):
    deg = degp_ref[0] + degp_ref[1] + 1.0
    dinv = lax.rsqrt(deg)
    dinv_ref[...] = dinv
    sn_ref[...] = dinv * dinv
    hw1 = jnp.dot(embp_ref[...], w1_ref[...], preferred_element_type=_f32)
    onehot = (x_ref[...] == lax.broadcasted_iota(_i32, (1, TP), 1)
              ).astype(_f32)
    h1w_ref[...] = jnp.dot(onehot, hw1, preferred_element_type=_f32)


def _tc_prep(degp, x2, embp, w1):
    return pl.pallas_call(
        _prep_body,
        out_shape=(
            jax.ShapeDtypeStruct((80, 128), _f32),
            jax.ShapeDtypeStruct((80, 128), _f32),
            jax.ShapeDtypeStruct((NP, D), _f32),
        ),
    )(degp, x2, embp, w1)


def _epi_body(m0_ref, m1_ref, hw_ref, sn_ref, b_ref, w_ref, out_ref):
    t = m0_ref[...] + m1_ref[...] + sn_ref[...] * hw_ref[...] + b_ref[...]
    h = jnp.maximum(t, 0.0)
    out_ref[...] = jnp.dot(h, w_ref[...], preferred_element_type=_f32)


def _tc_epi(m0, m1, hw, sn1, b, w):
    return pl.pallas_call(
        _epi_body,
        out_shape=jax.ShapeDtypeStruct((NP, D), _f32),
    )(m0, m1, hw, sn1, b, w)


def _pool_body(m0_ref, m1_ref, hw_ref, sn_ref, b_ref, batch_ref, out_ref):
    t = m0_ref[...] + m1_ref[...] + sn_ref[...] * hw_ref[...] + b_ref[...]
    onehot = (batch_ref[...] == lax.broadcasted_iota(_i32, (1, G), 1)
              ).astype(_f32)
    sums = lax.dot_general(onehot, t, (((0,), (0,)), ((), ())),
                           preferred_element_type=_f32)
    cnt = jnp.sum(onehot, axis=0)
    mean = sums / jnp.maximum(cnt, 1.0)[:, None]
    sos = jnp.sum(mean * mean, axis=1, keepdims=True)
    out_ref[...] = mean * lax.rsqrt(sos)


def _tc_pool(m0, m1, hw, sn1, b, batch2):
    return pl.pallas_call(
        _pool_body,
        out_shape=jax.ShapeDtypeStruct((G, D), _f32),
    )(m0, m1, hw, sn1, b, batch2)


def _shard_edges(a, fill):
    a = a.reshape(NW, EPT)
    pad = jnp.full((NW, EPTP - EPT), fill, a.dtype)
    return jnp.concatenate([a, pad], axis=1).reshape(NW, KC, CH)


def kernel(x, edge_index, edge_attr, batch, emb, W1, b1, W2, b2, W3, b3):
    src = edge_index[0].astype(_i32)
    dst = edge_index[1].astype(_i32)
    src3 = _shard_edges(src, 0)
    dst3 = _shard_edges(dst, 0)
    ew3 = _shard_edges(edge_attr, 0.0)

    x2 = jnp.concatenate(
        [x.astype(_i32), jnp.zeros((NP - N,), _i32)]).reshape(NP, 1)
    batch2 = jnp.concatenate(
        [batch.astype(_i32), jnp.full((NP - N,), G + 1, _i32)]).reshape(NP, 1)
    embp = jnp.concatenate([emb, jnp.zeros((TP - T, D), _f32)], axis=0)

    degp = _deg_kernel(dst3, ew3)
    dinv2, sn2, h1w = _tc_prep(degp.reshape(2, 80, 128), x2, embp, W1)
    dinv = dinv2.reshape(NP)
    sn1 = sn2.reshape(NP, 1)

    norm3 = _norm_kernel(src3, dst3, ew3, dinv)

    b1r = b1.reshape(1, D)
    b2r = b2.reshape(1, D)
    b3r = b3.reshape(1, D)

    msg1 = _mp_kernel(h1w, src3, dst3, norm3)
    h2w = _tc_epi(msg1[0], msg1[1], h1w, sn1, b1r, W2)
    msg2 = _mp_kernel(h2w, src3, dst3, norm3)
    h3w = _tc_epi(msg2[0], msg2[1], h2w, sn1, b2r, W3)
    msg3 = _mp_kernel(h3w, src3, dst3, norm3)
    return _tc_pool(msg3[0], msg3[1], h3w, sn1, b3r, batch2)
